# Initial kernel scaffold; baseline (speedup 1.0000x reference)
import jax
import jax.numpy as jnp
from jax.experimental import pallas as pl


def kernel(x, index):
    raise NotImplementedError("write your pallas kernel here")


# speedup vs baseline: 5.7840x; 5.7840x over previous
import functools

import jax
import jax.numpy as jnp
from jax import lax
from jax.experimental import pallas as pl
from jax.experimental.pallas import tpu as pltpu, tpu_sc as plsc

N_ROWS = 320000
D = 128
N_NODES = 10000
SCALE = 1.0 / (32.0 ** 0.5)

NC = 2
NS = 16
L = 16

COLS = D // NC
CHUNK = 512
SUB = 128
NSUB = CHUNK // SUB
N_CHUNKS = N_ROWS // CHUNK
CHUNKS_PER_TILE = N_CHUNKS // NS
OUT_ROWS_PER_TILE = N_NODES // NS
IDX_BLKS = N_ROWS // SUB


def _body(x_hbm, idx_hbm, out_hbm, xbuf, ibuf, obuf, acc, sem):
    c = lax.axis_index("c")
    s = lax.axis_index("s")
    col0 = c * COLS
    row0 = s * OUT_ROWS_PER_TILE

    z = jnp.zeros((L,), jnp.float32)

    def zero_row(r, carry):
        for jj in range(COLS // L):
            obuf[r, pl.ds(jj * L, L)] = z
        return carry

    lax.fori_loop(0, OUT_ROWS_PER_TILE, zero_row, 0)
    pltpu.sync_copy(obuf, acc.at[pl.ds(row0, OUT_ROWS_PER_TILE), :])
    plsc.subcore_barrier()

    base = s * CHUNKS_PER_TILE
    n_k = CHUNKS_PER_TILE + jnp.where(s == NS - 1, 1, 0)

    def chunk_body(k, carry):
        g = base + k
        r0 = g * CHUNK
        pltpu.sync_copy(x_hbm.at[pl.ds(r0, CHUNK), pl.ds(col0, COLS)], xbuf)
        pltpu.sync_copy(idx_hbm.at[pl.ds(g * NSUB, NSUB), :], ibuf)
        for j in range(NSUB):
            pltpu.sync_copy(
                xbuf.at[pl.ds(j * SUB, SUB), :],
                acc.at[ibuf.at[j]],
                add=True,
            )
        return carry

    lax.fori_loop(0, n_k, chunk_body, 0)
    plsc.subcore_barrier()

    pltpu.sync_copy(acc.at[pl.ds(row0, OUT_ROWS_PER_TILE), :], obuf)

    def scale_row(r, carry):
        for jj in range(COLS // L):
            v = obuf[r, pl.ds(jj * L, L)]
            obuf[r, pl.ds(jj * L, L)] = v * SCALE
        return carry

    lax.fori_loop(0, OUT_ROWS_PER_TILE, scale_row, 0)
    pltpu.sync_copy(
        obuf, out_hbm.at[pl.ds(row0, OUT_ROWS_PER_TILE), pl.ds(col0, COLS)]
    )


@jax.jit
def _scatter_scaled(x, idx2d):
    mesh = plsc.VectorSubcoreMesh(
        core_axis_name="c", subcore_axis_name="s", num_cores=NC, num_subcores=NS
    )
    return pl.kernel(
        _body,
        out_type=jax.ShapeDtypeStruct((N_NODES, D), jnp.float32),
        mesh=mesh,
        compiler_params=pltpu.CompilerParams(use_tc_tiling_on_sc=False),
        scratch_types=[
            pltpu.VMEM((CHUNK, COLS), jnp.float32),
            pltpu.VMEM((NSUB, SUB), jnp.int32),
            pltpu.VMEM((OUT_ROWS_PER_TILE, COLS), jnp.float32),
            pltpu.VMEM_SHARED((N_NODES, COLS), jnp.float32),
            pltpu.SemaphoreType.DMA,
        ],
    )(x, idx2d)


def kernel(x, index):
    idx2d = index.astype(jnp.int32).reshape(IDX_BLKS, SUB)
    return _scatter_scaled(x, idx2d)


# speedup vs baseline: 8.9721x; 1.5512x over previous
import jax
import jax.numpy as jnp
from jax import lax
from jax.experimental import pallas as pl
from jax.experimental.pallas import tpu as pltpu, tpu_sc as plsc

N_ROWS = 320000
D = 128
N_NODES = 10000
SCALE = 1.0 / (32.0 ** 0.5)

NC = 2
NS = 16
L = 16

COLS = D // NC
CHUNK = 256
SUB = 128
NSUB = CHUNK // SUB
N_CHUNKS = N_ROWS // CHUNK
CHUNKS_PER_TILE = N_CHUNKS // NS
OUT_ROWS_PER_TILE = N_NODES // NS
IDX_BLKS = N_ROWS // SUB


def _body(x_hbm, idx_hbm, out_hbm, xbuf, ibuf, obuf, acc, sem0, sem1):
    c = lax.axis_index("c")
    s = lax.axis_index("s")
    col0 = c * COLS
    row0 = s * OUT_ROWS_PER_TILE
    sems = (sem0, sem1)

    def issue_load(g, b):
        pltpu.async_copy(
            x_hbm.at[pl.ds(g * CHUNK, CHUNK), pl.ds(col0, COLS)],
            xbuf.at[b], sems[b])
        pltpu.async_copy(
            idx_hbm.at[pl.ds(g * NSUB, NSUB), :], ibuf.at[b], sems[b])

    def wait_load(b):
        pltpu.make_async_copy(
            x_hbm.at[pl.ds(0, CHUNK), pl.ds(0, COLS)], xbuf.at[b], sems[b]
        ).wait()
        pltpu.make_async_copy(
            idx_hbm.at[pl.ds(0, NSUB), :], ibuf.at[b], sems[b]
        ).wait()

    def scatter(b):
        for j in range(NSUB):
            pltpu.sync_copy(
                xbuf.at[b, pl.ds(j * SUB, SUB), :],
                acc.at[ibuf.at[b, j]],
                add=True,
            )

    z = jnp.zeros((L,), jnp.float32)

    def zero_row(r, carry):
        for jj in range(COLS // L):
            obuf[r, pl.ds(jj * L, L)] = z
        return carry

    lax.fori_loop(0, OUT_ROWS_PER_TILE, zero_row, 0)
    pltpu.sync_copy(obuf, acc.at[pl.ds(row0, OUT_ROWS_PER_TILE), :])
    plsc.subcore_barrier()

    base = s * CHUNKS_PER_TILE
    issue_load(base, 0)
    issue_load(base + 1, 1)

    @pl.loop(0, CHUNKS_PER_TILE, step=2)
    def _(k):
        for b in range(2):
            kk = k + b
            wait_load(b)
            scatter(b)

            @pl.when(kk + 2 < CHUNKS_PER_TILE)
            def _():
                issue_load(base + kk + 2, b)

    @pl.when(s >= NS - 2)
    def _():
        issue_load(N_CHUNKS - NS + s, 0)
        wait_load(0)
        scatter(0)

    plsc.subcore_barrier()

    pltpu.sync_copy(acc.at[pl.ds(row0, OUT_ROWS_PER_TILE), :], obuf)

    def scale_row(r, carry):
        for jj in range(COLS // L):
            v = obuf[r, pl.ds(jj * L, L)]
            obuf[r, pl.ds(jj * L, L)] = v * SCALE
        return carry

    lax.fori_loop(0, OUT_ROWS_PER_TILE, scale_row, 0)
    pltpu.sync_copy(
        obuf, out_hbm.at[pl.ds(row0, OUT_ROWS_PER_TILE), pl.ds(col0, COLS)]
    )


@jax.jit
def _scatter_scaled(x, idx2d):
    mesh = plsc.VectorSubcoreMesh(
        core_axis_name="c", subcore_axis_name="s", num_cores=NC, num_subcores=NS
    )
    return pl.kernel(
        _body,
        out_type=jax.ShapeDtypeStruct((N_NODES, D), jnp.float32),
        mesh=mesh,
        compiler_params=pltpu.CompilerParams(use_tc_tiling_on_sc=False),
        scratch_types=[
            pltpu.VMEM((2, CHUNK, COLS), jnp.float32),
            pltpu.VMEM((2, NSUB, SUB), jnp.int32),
            pltpu.VMEM((OUT_ROWS_PER_TILE, COLS), jnp.float32),
            pltpu.VMEM_SHARED((N_NODES, COLS), jnp.float32),
            pltpu.SemaphoreType.DMA,
            pltpu.SemaphoreType.DMA,
        ],
    )(x, idx2d)


def kernel(x, index):
    idx2d = index.astype(jnp.int32).reshape(IDX_BLKS, SUB)
    return _scatter_scaled(x, idx2d)
